# Initial kernel scaffold; baseline (speedup 1.0000x reference)
import functools
import jax
import jax.numpy as jnp
from jax import lax
from jax.experimental import pallas as pl
from jax.experimental.pallas import tpu as pltpu
from jax.experimental.pallas import tpu_sc as plsc

N = 10000
E = 320000
F = 128
C = 64

NC = 2
NS = 16
NW = NC * NS

BATCH = 128
CHUNKS = 80
EPW = CHUNKS * BATCH
E_PAD = NW * EPW

N_PAD = 10016
RPW = N_PAD // NS

_mesh = plsc.VectorSubcoreMesh(core_axis_name="c", subcore_axis_name="s")


def _zero_rows(buf, nrows, width16):
    z = jnp.zeros((16,), jnp.float32)

    @pl.loop(0, nrows)
    def _(i):
        for j in range(width16):
            buf[i, pl.ds(j * 16, 16)] = z


def _zero_stripe(src_v, dst_sh, s):
    base = s * RPW
    for k in range(4):
        pltpu.sync_copy(src_v, dst_sh.at[pl.ds(base + k * 128, 128)])
    pltpu.sync_copy(src_v.at[pl.ds(0, RPW - 512)],
                    dst_sh.at[pl.ds(base + 512, RPW - 512)])


@functools.partial(
    pl.kernel,
    out_type=jax.ShapeDtypeStruct((NC, N_PAD), jnp.float32),
    mesh=_mesh,
    scratch_types=[
        pltpu.VMEM((CHUNKS, BATCH), jnp.int32),
        pltpu.VMEM((BATCH,), jnp.float32),
        pltpu.VMEM((BATCH,), jnp.float32),
        pltpu.VMEM_SHARED((N_PAD,), jnp.float32),
    ],
)
def _deg_kernel(dst_hbm, degp_hbm, dst_v, ones_v, zero_v, deg_sh):
    c = lax.axis_index("c")
    s = lax.axis_index("s")
    wid = c * NS + s

    one = jnp.ones((16,), jnp.float32)
    z = jnp.zeros((16,), jnp.float32)
    for j in range(BATCH // 16):
        ones_v[pl.ds(j * 16, 16)] = one
        zero_v[pl.ds(j * 16, 16)] = z

    base = s * RPW
    for k in range(4):
        pltpu.sync_copy(zero_v, deg_sh.at[pl.ds(base + k * 128, 128)])
    pltpu.sync_copy(zero_v.at[pl.ds(0, RPW - 512)],
                    deg_sh.at[pl.ds(base + 512, RPW - 512)])

    pltpu.sync_copy(dst_hbm.at[wid], dst_v)
    plsc.subcore_barrier()

    @pl.loop(0, CHUNKS)
    def _(j):
        pltpu.sync_copy(ones_v, deg_sh.at[dst_v.at[j]], add=True)

    plsc.subcore_barrier()
    pltpu.sync_copy(deg_sh.at[pl.ds(base, RPW)],
                    degp_hbm.at[c, pl.ds(base, RPW)])


@functools.partial(
    pl.kernel,
    out_type=jax.ShapeDtypeStruct((NC, N_PAD, C), jnp.float32),
    mesh=_mesh,
    scratch_types=[
        pltpu.VMEM((CHUNKS, BATCH), jnp.int32),
        pltpu.VMEM((CHUNKS, BATCH), jnp.int32),
        pltpu.VMEM((BATCH, C), jnp.float32),
        pltpu.VMEM((BATCH, C), jnp.float32),
        pltpu.VMEM_SHARED((N_PAD, C), jnp.float32),
        pltpu.SemaphoreType.DMA,
        pltpu.SemaphoreType.DMA,
    ],
)
def _agg_kernel(h2_hbm, src_hbm, dst_hbm, accp_hbm,
                src_v, dst_v, rows0, rows1, acc_sh, sem0, sem1):
    c = lax.axis_index("c")
    s = lax.axis_index("s")
    wid = c * NS + s

    _zero_rows(rows0, BATCH, C // 16)
    _zero_stripe(rows0, acc_sh, s)

    pltpu.sync_copy(src_hbm.at[wid], src_v)
    pltpu.sync_copy(dst_hbm.at[wid], dst_v)
    plsc.subcore_barrier()

    bufs = (rows0, rows1)
    sems = (sem0, sem1)
    pltpu.async_copy(h2_hbm.at[src_v.at[0]], bufs[0], sems[0])

    @pl.loop(0, CHUNKS, step=2)
    def _(j):
        for b in range(2):
            jj = j + b
            nxt = jj + 1

            @pl.when(nxt < CHUNKS)
            def _():
                pltpu.async_copy(h2_hbm.at[src_v.at[nxt]], bufs[1 - b], sems[1 - b])

            pltpu.make_async_copy(h2_hbm.at[src_v.at[jj]], bufs[b], sems[b]).wait()
            pltpu.sync_copy(bufs[b], acc_sh.at[dst_v.at[jj]], add=True)

    plsc.subcore_barrier()
    base = s * RPW
    pltpu.sync_copy(acc_sh.at[pl.ds(base, RPW)],
                    accp_hbm.at[c].at[pl.ds(base, RPW)])


RB = 1000


def _h2_body(deg_ref, x_ref, w_ref, h2_ref):
    i = pl.program_id(0)
    deg = deg_ref[0] + deg_ref[1] + 1.0
    dseg = lax.dynamic_slice(deg, (i * RB,), (RB,))
    dinv = lax.rsqrt(dseg)
    h = jnp.dot(x_ref[...], w_ref[...], preferred_element_type=jnp.float32)
    h2_ref[...] = h * dinv[:, None]


def _fin_body(deg_ref, acc_ref, h2_ref, b_ref, out_ref):
    i = pl.program_id(0)
    deg = deg_ref[0] + deg_ref[1] + 1.0
    dseg = lax.dynamic_slice(deg, (i * RB,), (RB,))
    dinv = lax.rsqrt(dseg)
    tot = acc_ref[0] + acc_ref[1] + h2_ref[...]
    out_ref[...] = tot * dinv[:, None] + b_ref[...]


def _tc_h2(degp, x, W):
    return pl.pallas_call(
        _h2_body,
        grid=(N // RB,),
        in_specs=[
            pl.BlockSpec((NC, N_PAD), lambda i: (0, 0)),
            pl.BlockSpec((RB, F), lambda i: (i, 0)),
            pl.BlockSpec((F, C), lambda i: (0, 0)),
        ],
        out_specs=pl.BlockSpec((RB, C), lambda i: (i, 0)),
        out_shape=jax.ShapeDtypeStruct((N, C), jnp.float32),
    )(degp, x, W)


def _tc_final(degp, accp, h2, b):
    return pl.pallas_call(
        _fin_body,
        grid=(N // RB,),
        in_specs=[
            pl.BlockSpec((NC, N_PAD), lambda i: (0, 0)),
            pl.BlockSpec((NC, RB, C), lambda i: (0, i, 0)),
            pl.BlockSpec((RB, C), lambda i: (i, 0)),
            pl.BlockSpec((1, C), lambda i: (0, 0)),
        ],
        out_specs=pl.BlockSpec((RB, C), lambda i: (i, 0)),
        out_shape=jax.ShapeDtypeStruct((N, C), jnp.float32),
    )(degp, accp, h2, b)


def kernel(x, adj, W, b):
    src = adj[0].astype(jnp.int32)
    dst = adj[1].astype(jnp.int32)
    pad = E_PAD - E
    src3 = jnp.concatenate([src, jnp.zeros((pad,), jnp.int32)]).reshape(NW, CHUNKS, BATCH)
    dst3 = jnp.concatenate([dst, jnp.full((pad,), N, jnp.int32)]).reshape(NW, CHUNKS, BATCH)

    degp = _deg_kernel(dst3)
    h2 = _tc_h2(degp, x, W)
    accp = _agg_kernel(h2, src3, dst3)
    accp = accp[:, :N, :]
    return _tc_final(degp, accp, h2, b.reshape(1, C))


# speedup vs baseline: 26.0048x; 26.0048x over previous
import functools
import jax
import jax.numpy as jnp
from jax import lax
from jax.experimental import pallas as pl
from jax.experimental.pallas import tpu as pltpu
from jax.experimental.pallas import tpu_sc as plsc

N = 10000
E = 320000
F = 128
C = 64

NC = 2
NS = 16
NW = NC * NS

BATCH = 128
CHUNKS = 80
EPW = CHUNKS * BATCH
E_PAD = NW * EPW

N_PAD = 10240
RPW = N_PAD // NS

_mesh = plsc.VectorSubcoreMesh(core_axis_name="c", subcore_axis_name="s")


@functools.partial(
    pl.kernel,
    out_type=jax.ShapeDtypeStruct((NC, N_PAD), jnp.float32),
    mesh=_mesh,
    scratch_types=[
        pltpu.VMEM((CHUNKS, BATCH), jnp.int32),
        pltpu.VMEM((BATCH,), jnp.float32),
        pltpu.VMEM((BATCH,), jnp.float32),
        pltpu.VMEM_SHARED((N_PAD,), jnp.float32),
    ],
)
def _deg_kernel(dst_hbm, degp_hbm, dst_v, ones_v, zero_v, deg_sh):
    c = lax.axis_index("c")
    s = lax.axis_index("s")
    wid = c * NS + s

    one = jnp.ones((16,), jnp.float32)
    z = jnp.zeros((16,), jnp.float32)
    for j in range(BATCH // 16):
        ones_v[pl.ds(j * 16, 16)] = one
        zero_v[pl.ds(j * 16, 16)] = z

    base = s * RPW
    for k in range(RPW // BATCH):
        pltpu.sync_copy(zero_v, deg_sh.at[pl.ds(base + k * BATCH, BATCH)])

    pltpu.sync_copy(dst_hbm.at[wid], dst_v)
    plsc.subcore_barrier()

    @pl.loop(0, CHUNKS)
    def _(j):
        pltpu.sync_copy(ones_v, deg_sh.at[dst_v.at[j]], add=True)

    plsc.subcore_barrier()
    pltpu.sync_copy(deg_sh.at[pl.ds(base, RPW)],
                    degp_hbm.at[c, pl.ds(base, RPW)])


@functools.partial(
    pl.kernel,
    out_type=jax.ShapeDtypeStruct((NC, N_PAD, C), jnp.float32),
    mesh=_mesh,
    scratch_types=[
        pltpu.VMEM((CHUNKS, BATCH), jnp.int32),
        pltpu.VMEM((CHUNKS, BATCH), jnp.int32),
        pltpu.VMEM((BATCH, C), jnp.float32),
        pltpu.VMEM((BATCH, C), jnp.float32),
        pltpu.VMEM_SHARED((N_PAD, C), jnp.float32),
        pltpu.SemaphoreType.DMA,
        pltpu.SemaphoreType.DMA,
    ],
    compiler_params=pltpu.CompilerParams(use_tc_tiling_on_sc=False),
)
def _agg_kernel(h2_hbm, src_hbm, dst_hbm, accp_hbm,
                src_v, dst_v, rows0, rows1, acc_sh, sem0, sem1):
    c = lax.axis_index("c")
    s = lax.axis_index("s")
    wid = c * NS + s

    z = jnp.zeros((16,), jnp.float32)

    @pl.loop(0, BATCH)
    def _(i):
        for j in range(C // 16):
            rows0[i, pl.ds(j * 16, 16)] = z

    base = s * RPW
    for k in range(RPW // BATCH):
        pltpu.sync_copy(rows0, acc_sh.at[pl.ds(base + k * BATCH, BATCH)])

    pltpu.sync_copy(src_hbm.at[wid], src_v)
    pltpu.sync_copy(dst_hbm.at[wid], dst_v)
    plsc.subcore_barrier()

    bufs = (rows0, rows1)
    sems = (sem0, sem1)
    pltpu.async_copy(h2_hbm.at[src_v.at[0]], bufs[0], sems[0])

    @pl.loop(0, CHUNKS, step=2)
    def _(j):
        for b in range(2):
            jj = j + b
            nxt = jj + 1

            @pl.when(nxt < CHUNKS)
            def _():
                pltpu.async_copy(h2_hbm.at[src_v.at[nxt]], bufs[1 - b], sems[1 - b])

            pltpu.make_async_copy(h2_hbm.at[src_v.at[jj]], bufs[b], sems[b]).wait()
            pltpu.sync_copy(bufs[b], acc_sh.at[dst_v.at[jj]], add=True)

    plsc.subcore_barrier()
    pltpu.sync_copy(acc_sh.at[pl.ds(base, RPW)],
                    accp_hbm.at[c, pl.ds(base, RPW)])


RB = 1024


def _h2_body(deg_ref, x_ref, w_ref, h2_ref):
    deg = deg_ref[0] + deg_ref[1] + 1.0
    dinv = lax.rsqrt(deg)
    h = jnp.dot(x_ref[...], w_ref[...], preferred_element_type=jnp.float32)
    h2_ref[...] = h * dinv[:, None]


def _fin_body(deg_ref, acc_ref, h2_ref, b_ref, out_ref):
    deg = deg_ref[0] + deg_ref[1] + 1.0
    dinv = lax.rsqrt(deg)
    tot = acc_ref[0] + acc_ref[1] + h2_ref[...]
    out_ref[...] = tot * dinv[:, None] + b_ref[...]


def _tc_h2(degp, x_pad, W):
    return pl.pallas_call(
        _h2_body,
        grid=(N_PAD // RB,),
        in_specs=[
            pl.BlockSpec((NC, RB), lambda i: (0, i)),
            pl.BlockSpec((RB, F), lambda i: (i, 0)),
            pl.BlockSpec((F, C), lambda i: (0, 0)),
        ],
        out_specs=pl.BlockSpec((RB, C), lambda i: (i, 0)),
        out_shape=jax.ShapeDtypeStruct((N_PAD, C), jnp.float32),
    )(degp, x_pad, W)


def _tc_final(degp, accp, h2, b):
    return pl.pallas_call(
        _fin_body,
        grid=(N_PAD // RB,),
        in_specs=[
            pl.BlockSpec((NC, RB), lambda i: (0, i)),
            pl.BlockSpec((NC, RB, C), lambda i: (0, i, 0)),
            pl.BlockSpec((RB, C), lambda i: (i, 0)),
            pl.BlockSpec((1, C), lambda i: (0, 0)),
        ],
        out_specs=pl.BlockSpec((RB, C), lambda i: (i, 0)),
        out_shape=jax.ShapeDtypeStruct((N_PAD, C), jnp.float32),
    )(degp, accp, h2, b)


def kernel(x, adj, W, b):
    src = adj[0].astype(jnp.int32)
    dst = adj[1].astype(jnp.int32)
    pad = E_PAD - E
    src3 = jnp.concatenate([src, jnp.zeros((pad,), jnp.int32)]).reshape(NW, CHUNKS, BATCH)
    dst3 = jnp.concatenate([dst, jnp.full((pad,), N, jnp.int32)]).reshape(NW, CHUNKS, BATCH)
    x_pad = jnp.concatenate([x, jnp.zeros((N_PAD - N, F), jnp.float32)])

    degp = _deg_kernel(dst3)
    h2 = _tc_h2(degp, x_pad, W)
    accp = _agg_kernel(h2, src3, dst3)
    out = _tc_final(degp, accp, h2, b.reshape(1, C))
    return out[:N]
